# Initial kernel scaffold; baseline (speedup 1.0000x reference)
import jax
import jax.numpy as jnp
from jax.experimental import pallas as pl


def kernel(x_question, x_answer, pos_edge_label_index, neg_edge_label_index):
    raise NotImplementedError("write your pallas kernel here")


# speedup vs baseline: 4.4182x; 4.4182x over previous
import functools

import jax
import jax.numpy as jnp
from jax import lax
from jax.experimental import pallas as pl
from jax.experimental.pallas import tpu as pltpu, tpu_sc as plsc

N_NODES = 50000
N_EDGES = 800000
CH = 64
NC, NS = 2, 16
NW = NC * NS
PER_TILE = N_EDGES // NW
C = 200
NCHUNK = PER_TILE // C
UNROLL = 8


def _dot_chunk(qrows, arows, outv):
    iota = lax.broadcasted_iota(jnp.int32, (16,), 0)
    lane0 = iota == 0
    ones = jnp.full((16,), 1, jnp.int32)

    def body(i, _):
        e0 = i * UNROLL
        for l in range(UNROLL):
            e = e0 + l
            acc = qrows[e, pl.ds(0, 16)] * arows[e, pl.ds(0, 16)]
            for j in range(1, CH // 16):
                acc = acc + qrows[e, pl.ds(j * 16, 16)] * arows[e, pl.ds(j * 16, 16)]
            for sh in (8, 4, 2, 1):
                acc = acc + jnp.take(acc, iota ^ sh)
            plsc.store_scatter(outv, [ones * e], acc, mask=lane0)
        return 0

    lax.fori_loop(0, C // UNROLL, body, 0)


def _body(xq, xa, p0, p1, n0, n1, pos_out, neg_out,
          idxq, idxa, qrows, arows, outv, semq, sema):
    wid = lax.axis_index("s") * NC + lax.axis_index("c")
    base = wid * PER_TILE

    def make_chunk(src0, src1, out_hbm):
        def chunk(i, _):
            off = base + i * C
            pltpu.sync_copy(src0.at[pl.ds(off, C)], idxq)
            pltpu.sync_copy(src1.at[pl.ds(off, C)], idxa)
            cq = pltpu.async_copy(xq.at[idxq], qrows, semq)
            ca = pltpu.async_copy(xa.at[idxa], arows, sema)
            cq.wait()
            ca.wait()
            _dot_chunk(qrows, arows, outv)
            pltpu.sync_copy(outv, out_hbm.at[pl.ds(off, C)])
            return 0
        return chunk

    lax.fori_loop(0, NCHUNK, make_chunk(p0, p1, pos_out), 0)
    lax.fori_loop(0, NCHUNK, make_chunk(n0, n1, neg_out), 0)


@functools.partial(
    pl.kernel,
    out_type=(
        jax.ShapeDtypeStruct((N_EDGES,), jnp.float32),
        jax.ShapeDtypeStruct((N_EDGES,), jnp.float32),
    ),
    mesh=plsc.VectorSubcoreMesh(core_axis_name="c", subcore_axis_name="s"),
    compiler_params=pltpu.CompilerParams(
        needs_layout_passes=False,
        use_tc_tiling_on_sc=False,
    ),
    scratch_types=[
        pltpu.VMEM((C,), jnp.int32),
        pltpu.VMEM((C,), jnp.int32),
        pltpu.VMEM((C, CH), jnp.float32),
        pltpu.VMEM((C, CH), jnp.float32),
        pltpu.VMEM((C,), jnp.float32),
        pltpu.SemaphoreType.DMA,
        pltpu.SemaphoreType.DMA,
    ],
)
def _sc_link_pred(*args):
    _body(*args)


def kernel(x_question, x_answer, pos_edge_label_index, neg_edge_label_index):
    pos = pos_edge_label_index.astype(jnp.int32)
    neg = neg_edge_label_index.astype(jnp.int32)
    return _sc_link_pred(x_question, x_answer,
                         pos[0], pos[1], neg[0], neg[1])
